# Initial kernel scaffold; baseline (speedup 1.0000x reference)
import functools

import jax
import jax.numpy as jnp
from jax import lax
from jax.experimental import pallas as pl
from jax.experimental.pallas import tpu as pltpu
from jax.experimental.pallas import tpu_sc as plsc

F = 26
V = 100000
D = 32
B = 16384
TOT = B * F
NC, NS, L = 2, 16, 16
NW = NC * NS
PER_W = TOT // NW
CHUNK = 128
NCH = PER_W // CHUNK


def _emb_body(x_hbm, tab_hbm, out_hbm, xbuf, idxbuf, rowbuf, sem):
    wid = lax.axis_index("s") * NC + lax.axis_index("c")
    base = wid * PER_W

    def step(g, carry):
        off = base + g * CHUNK
        pltpu.sync_copy(x_hbm.at[pl.ds(off, CHUNK)], xbuf)
        for v in range(CHUNK // L):
            pos = off + v * L + lax.iota(jnp.int32, L)
            fld = lax.rem(pos, F)
            idxbuf[pl.ds(v * L, L)] = xbuf[pl.ds(v * L, L)] + fld * V
        pltpu.async_copy(tab_hbm.at[idxbuf], rowbuf, sem).wait()
        pltpu.sync_copy(rowbuf, out_hbm.at[pl.ds(off, CHUNK)])
        return carry

    lax.fori_loop(0, NCH, step, 0)


@jax.jit
def _multi_embedding(x_flat, tab_flat):
    mesh = plsc.VectorSubcoreMesh(core_axis_name="c", subcore_axis_name="s")
    run = functools.partial(
        pl.kernel,
        mesh=mesh,
        out_type=jax.ShapeDtypeStruct((TOT, D), jnp.float32),
        scratch_types=[
            pltpu.VMEM((CHUNK,), jnp.int32),
            pltpu.VMEM((CHUNK,), jnp.int32),
            pltpu.VMEM((CHUNK, D), jnp.float32),
            pltpu.SemaphoreType.DMA,
        ],
    )(_emb_body)
    return run(x_flat, tab_flat)


def kernel(x, tables):
    x_flat = x.reshape(TOT)
    tab_flat = tables.reshape(F * V, D)
    out = _multi_embedding(x_flat, tab_flat)
    return out.reshape(B, F * D)


# speedup vs baseline: 1.1144x; 1.1144x over previous
import functools

import jax
import jax.numpy as jnp
from jax import lax
from jax.experimental import pallas as pl
from jax.experimental.pallas import tpu as pltpu
from jax.experimental.pallas import tpu_sc as plsc

F = 26
V = 100000
D = 32
B = 16384
TOT = B * F
NC, NS, L = 2, 16, 16
NW = NC * NS
PER_W = TOT // NW
CHUNK = 128
NCH = PER_W // CHUNK


def _emb_body(x_hbm, tab_hbm, out_hbm, xbuf, idxbuf, rowbuf, sem):
    wid = lax.axis_index("s") * NC + lax.axis_index("c")
    base = wid * PER_W

    def step(g, carry):
        off = base + g * CHUNK
        pltpu.sync_copy(x_hbm.at[pl.ds(off, CHUNK)], xbuf)
        for v in range(CHUNK // L):
            pos = off + v * L + lax.iota(jnp.int32, L)
            fld = lax.rem(pos, F)
            idxbuf[pl.ds(v * L, L)] = xbuf[pl.ds(v * L, L)] + fld * V
        pltpu.async_copy(tab_hbm.at[idxbuf], rowbuf, sem).wait()
        pltpu.sync_copy(rowbuf, out_hbm.at[pl.ds(off, CHUNK)])
        return carry

    lax.fori_loop(0, NCH, step, 0)


@jax.jit
def _multi_embedding(x_flat, tab_flat):
    mesh = plsc.VectorSubcoreMesh(core_axis_name="c", subcore_axis_name="s")
    run = functools.partial(
        pl.kernel,
        mesh=mesh,
        compiler_params=pltpu.CompilerParams(use_tc_tiling_on_sc=False),
        out_type=jax.ShapeDtypeStruct((TOT, D), jnp.float32),
        scratch_types=[
            pltpu.VMEM((CHUNK,), jnp.int32),
            pltpu.VMEM((CHUNK,), jnp.int32),
            pltpu.VMEM((CHUNK, D), jnp.float32),
            pltpu.SemaphoreType.DMA,
        ],
    )(_emb_body)
    return run(x_flat, tab_flat)


def kernel(x, tables):
    x_flat = x.reshape(TOT)
    tab_flat = tables.reshape(F * V, D)
    out = _multi_embedding(x_flat, tab_flat)
    return out.reshape(B, F * D)


# speedup vs baseline: 1.2148x; 1.0901x over previous
import functools

import jax
import jax.numpy as jnp
from jax import lax
from jax.experimental import pallas as pl
from jax.experimental.pallas import tpu as pltpu
from jax.experimental.pallas import tpu_sc as plsc

F = 26
V = 100000
D = 32
B = 16384
TOT = B * F
NC, NS, L = 2, 16, 16
NW = NC * NS
PER_W = TOT // NW
IDXW = 128
ROWS_W = PER_W // IDXW
SUPR = 13
SUP = SUPR * IDXW
NSUP = ROWS_W // SUPR


def _emb_body(x_hbm, tab_hbm, out_hbm, idxbuf, rb0, rb1, gs0, gs1, cs0, cs1):
    wid = lax.axis_index("s") * NC + lax.axis_index("c")
    base = wid * PER_W

    pltpu.sync_copy(x_hbm.at[pl.ds(base, PER_W)], idxbuf)

    def compute_idx(s):
        def row(r, c):
            for v in range(IDXW // L):
                pos = base + r * IDXW + (v * L + lax.iota(jnp.int32, L))
                sl = pl.ds(r * IDXW + v * L, L)
                idxbuf[sl] = idxbuf[sl] + lax.rem(pos, F) * V
            return c
        lax.fori_loop(s * SUPR, (s + 1) * SUPR, row, 0)

    def gather(s, rb, sem):
        return pltpu.make_async_copy(
            tab_hbm.at[idxbuf.at[pl.ds(s * SUP, SUP)]], rb, sem)

    def out_copy(s, rb, sem):
        return pltpu.make_async_copy(
            rb, out_hbm.at[pl.ds(base + s * SUP, SUP)], sem)

    compute_idx(0)
    gather(0, rb0, gs0).start()
    compute_idx(1)
    gather(1, rb1, gs1).start()

    for s in range(NSUP):
        rb, gs, cs = (rb0, gs0, cs0) if s % 2 == 0 else (rb1, gs1, cs1)
        gather(s, rb, gs).wait()
        out_copy(s, rb, cs).start()
        if s + 2 < NSUP:
            compute_idx(s + 2)
            out_copy(s, rb, cs).wait()
            gather(s + 2, rb, gs).start()

    out_copy(NSUP - 2, rb0, cs0).wait()
    out_copy(NSUP - 1, rb1, cs1).wait()


@jax.jit
def _multi_embedding(x2d, tab_flat):
    mesh = plsc.VectorSubcoreMesh(core_axis_name="c", subcore_axis_name="s")
    run = functools.partial(
        pl.kernel,
        mesh=mesh,
        compiler_params=pltpu.CompilerParams(use_tc_tiling_on_sc=False),
        out_type=jax.ShapeDtypeStruct((TOT, D), jnp.float32),
        scratch_types=[
            pltpu.VMEM((PER_W,), jnp.int32),
            pltpu.VMEM((SUP, D), jnp.float32),
            pltpu.VMEM((SUP, D), jnp.float32),
            pltpu.SemaphoreType.DMA,
            pltpu.SemaphoreType.DMA,
            pltpu.SemaphoreType.DMA,
            pltpu.SemaphoreType.DMA,
        ],
    )(_emb_body)
    return run(x2d, tab_flat)


def kernel(x, tables):
    x2d = x.reshape(TOT)
    tab_flat = tables.reshape(F * V, D)
    out = _multi_embedding(x2d, tab_flat)
    return out.reshape(B, F * D)
